# Initial kernel scaffold; baseline (speedup 1.0000x reference)
import jax
import jax.numpy as jnp
from jax.experimental import pallas as pl


def kernel(x, params):
    raise NotImplementedError("write your pallas kernel here")


# speedup vs baseline: 1.4426x; 1.4426x over previous
import functools

import jax
import jax.numpy as jnp
from jax.experimental import pallas as pl

P = 14
NH = 12
TOPK = 2
NE = 7
B = 4
S = 256
D = 768
FF = 3072
DH = D // NH
KPAD = 640
NLANE = 128
FB = 768
NF = FF // FB


def _ln(x, g, b):
    m = jnp.mean(x, axis=-1, keepdims=True)
    v = jnp.mean((x - m) * (x - m), axis=-1, keepdims=True)
    return (x - m) * jax.lax.rsqrt(v + 1e-5) * g + b


def _enc_body(pat_ref, pw_ref, pb_ref, pos_ref, g1_ref, b1_ref,
              wq_ref, bq_ref, wk_ref, bk_ref, wv_ref, bv_ref,
              wo_ref, bo_ref, h_ref, attn_ref):
    pat = pat_ref[0]
    h0 = jnp.dot(pat, pw_ref[...], preferred_element_type=jnp.float32)
    h0 = h0 + pb_ref[...] + pos_ref[...]
    xl = _ln(h0, g1_ref[...], b1_ref[...])
    q = jnp.dot(xl, wq_ref[...], preferred_element_type=jnp.float32) + bq_ref[...]
    k = jnp.dot(xl, wk_ref[...], preferred_element_type=jnp.float32) + bk_ref[...]
    v = jnp.dot(xl, wv_ref[...], preferred_element_type=jnp.float32) + bv_ref[...]
    scale = 1.0 / (DH ** 0.5)
    outs = []
    for hh in range(NH):
        sl = slice(hh * DH, (hh + 1) * DH)
        qh, kh, vh = q[:, sl], k[:, sl], v[:, sl]
        sc = jax.lax.dot_general(qh, kh, (((1,), (1,)), ((), ())),
                                 preferred_element_type=jnp.float32) * scale
        mx = jnp.max(sc, axis=-1, keepdims=True)
        e = jnp.exp(sc - mx)
        pr = e / jnp.sum(e, axis=-1, keepdims=True)
        attn_ref[0, hh] = pr
        outs.append(jnp.dot(pr, vh, preferred_element_type=jnp.float32))
    sa = jnp.concatenate(outs, axis=1)
    sa = jnp.dot(sa, wo_ref[...], preferred_element_type=jnp.float32) + bo_ref[...]
    h_ref[0] = h0 + sa


def _encoder(patches, pw, pb, pos, g1, b1, wq, bq, wk, bk, wv, bv, wo, bo):
    full = lambda shp: pl.BlockSpec(shp, lambda i: (0,) * len(shp))
    return pl.pallas_call(
        _enc_body,
        grid=(B,),
        in_specs=[
            pl.BlockSpec((1, S, KPAD), lambda i: (i, 0, 0)),
            full((KPAD, D)), full((1, D)), full((S, D)),
            full((1, D)), full((1, D)),
            full((D, D)), full((1, D)), full((D, D)), full((1, D)),
            full((D, D)), full((1, D)), full((D, D)), full((1, D)),
        ],
        out_specs=[
            pl.BlockSpec((1, S, D), lambda i: (i, 0, 0)),
            pl.BlockSpec((1, NH, S, S), lambda i: (i, 0, 0, 0)),
        ],
        out_shape=[
            jax.ShapeDtypeStruct((B, S, D), jnp.float32),
            jax.ShapeDtypeStruct((B, NH, S, S), jnp.float32),
        ],
    )(patches, pw, pb, pos, g1, b1, wq, bq, wk, bk, wv, bv, wo, bo)


def _router_body(h_ref, g2_ref, b2_ref, wr_ref, br_ref,
                 masks_ref, loss_ref, m128_ref):
    h = h_ref[...]
    rin = _ln(h, g2_ref[...], b2_ref[...])
    logits = jnp.dot(rin, wr_ref[...], preferred_element_type=jnp.float32) + br_ref[...]
    lane = jax.lax.broadcasted_iota(jnp.int32, (B * S, NLANE), 1)
    neg = jnp.float32(-1e30)
    logits = jnp.where(lane < NE, logits, neg)
    mx = jnp.max(logits, axis=-1, keepdims=True)
    e = jnp.exp(logits - mx)
    probs = e / jnp.sum(e, axis=-1, keepdims=True)
    v1 = jnp.max(probs, axis=-1, keepdims=True)
    i1 = jnp.argmax(probs, axis=-1).reshape(B * S, 1)
    p2 = jnp.where(lane == i1, -1.0, probs)
    v2 = jnp.max(p2, axis=-1, keepdims=True)
    i2 = jnp.argmax(p2, axis=-1).reshape(B * S, 1)
    wsum = v1 + v2
    w1 = v1 / wsum
    w2 = v2 / wsum
    m128 = jnp.where(lane == i1, w1, 0.0) + jnp.where(lane == i2, w2, 0.0)
    masks_ref[...] = m128[:, :NE]
    m128_ref[...] = m128
    imp = jnp.mean(probs, axis=0, keepdims=True)
    load = jnp.mean((m128 > 0).astype(jnp.float32), axis=0, keepdims=True)
    loss = jnp.float32(NE) * jnp.sum(imp * load, axis=-1, keepdims=True)
    loss_ref[...] = jnp.broadcast_to(loss, (1, NLANE))


def _router(h2d, g2, b2, wr, br):
    full = lambda shp: pl.BlockSpec(shp, lambda: (0,) * len(shp))
    return pl.pallas_call(
        _router_body,
        in_specs=[full((B * S, D)), full((1, D)), full((1, D)),
                  full((D, NLANE)), full((1, NLANE))],
        out_specs=[full((B * S, NE)), full((1, NLANE)), full((B * S, NLANE))],
        out_shape=[
            jax.ShapeDtypeStruct((B * S, NE), jnp.float32),
            jax.ShapeDtypeStruct((1, NLANE), jnp.float32),
            jax.ShapeDtypeStruct((B * S, NLANE), jnp.float32),
        ],
    )(h2d, g2, b2, wr, br)


def _exp_body(h_ref, m128_ref, w1_ref, b1_ref, w2_ref, b2_ref, acc_ref):
    e = pl.program_id(0)
    f = pl.program_id(1)

    @pl.when(jnp.logical_and(e == 0, f == 0))
    def _():
        acc_ref[...] = jnp.zeros_like(acc_ref)

    x = h_ref[...]
    h1 = jnp.dot(x, w1_ref[0], preferred_element_type=jnp.float32) + b1_ref[0]
    g = jax.nn.gelu(h1)
    part = jnp.dot(g, w2_ref[0], preferred_element_type=jnp.float32)
    lane = jax.lax.broadcasted_iota(jnp.int32, (B * S, NLANE), 1)
    sel = jnp.where(lane == e, m128_ref[...], 0.0)
    me = jnp.sum(sel, axis=-1, keepdims=True)
    bias = jnp.where(f == 0, 1.0, 0.0) * b2_ref[0]
    acc_ref[...] += me * (part + bias)


def _experts(h2d, m128, ew1, eb1, ew2, eb2):
    full = lambda shp: pl.BlockSpec(shp, lambda e, f: (0,) * len(shp))
    return pl.pallas_call(
        _exp_body,
        grid=(NE, NF),
        in_specs=[
            full((B * S, D)), full((B * S, NLANE)),
            pl.BlockSpec((1, D, FB), lambda e, f: (e, 0, f)),
            pl.BlockSpec((1, 1, FB), lambda e, f: (e, 0, f)),
            pl.BlockSpec((1, FB, D), lambda e, f: (e, f, 0)),
            pl.BlockSpec((1, 1, D), lambda e, f: (e, 0, 0)),
        ],
        out_specs=pl.BlockSpec((B * S, D), lambda e, f: (0, 0)),
        out_shape=jax.ShapeDtypeStruct((B * S, D), jnp.float32),
    )(h2d, m128, ew1, eb1, ew2, eb2)


def _fin_body(acc_ref, g3_ref, b3_ref, cw_ref, cb_ref, fv_ref, cls_ref):
    a = acc_ref[...]
    o = _ln(a, g3_ref[...], b3_ref[...])
    fv = jnp.mean(o, axis=1)
    fv_ref[...] = fv
    cls_ref[...] = jnp.dot(fv, cw_ref[...], preferred_element_type=jnp.float32) + cb_ref[...]


def _final(acc3, g3, b3, cw, cb):
    full = lambda shp: pl.BlockSpec(shp, lambda: (0,) * len(shp))
    return pl.pallas_call(
        _fin_body,
        in_specs=[full((B, S, D)), full((1, D)), full((1, D)),
                  full((D, D)), full((1, D))],
        out_specs=[full((B, D)), full((B, D))],
        out_shape=[jax.ShapeDtypeStruct((B, D), jnp.float32),
                   jax.ShapeDtypeStruct((B, D), jnp.float32)],
    )(acc3, g3, b3, cw, cb)


def kernel(x, params):
    p = params
    hp = wp = 224 // P
    patches = x.reshape(B, 3, hp, P, wp, P).transpose(0, 2, 4, 3, 5, 1)
    patches = patches.reshape(B, S, P * P * 3)
    patches = jnp.pad(patches, ((0, 0), (0, 0), (0, KPAD - P * P * 3)))
    pw = jnp.pad(p['patch_W'], ((0, KPAD - P * P * 3), (0, 0)))
    row = lambda a: a.reshape(1, -1)
    pos = p['pos'].reshape(S, D)

    h, attn = _encoder(patches, pw, row(p['patch_b']), pos,
                       row(p['ln1_g']), row(p['ln1_b']),
                       p['Wq'], row(p['bq']), p['Wk'], row(p['bk']),
                       p['Wv'], row(p['bv']), p['Wo'], row(p['bo']))

    h2d = h.reshape(B * S, D)
    wr = jnp.pad(p['router_W'], ((0, 0), (0, NLANE - NE)))
    br = jnp.pad(p['router_b'], ((0, NLANE - NE))).reshape(1, NLANE)
    masks2d, loss_row, m128 = _router(h2d, row(p['ln2_g']), row(p['ln2_b']), wr, br)

    acc = _experts(h2d, m128,
                   p['exp_W1'], p['exp_b1'].reshape(NE, 1, FF),
                   p['exp_W2'], p['exp_b2'].reshape(NE, 1, D))

    fv, cls = _final(acc.reshape(B, S, D), row(p['ln3_g']), row(p['ln3_b']),
                     p['cls_W'], row(p['cls_b']))

    masks = masks2d.reshape(B, S, NE)
    router_loss = loss_row[0, 0]
    return fv, cls, router_loss, masks, attn
